# baseline (device time: 55247 ns/iter reference)
import jax
import jax.numpy as jnp
from jax import lax
from jax.experimental import pallas as pl
from jax.experimental.pallas import tpu as pltpu

N_DEV = 32
M_PER = 128
CHUNK = 4


def kernel(x, w_mat):
    m_glob, k_per = x.shape
    k_glob, n = w_mat.shape

    def body(x_ref, w_ref, out_ref, gathered_ref, send_sems, recv_sems):
        me = lax.axis_index("i")

        gathered_ref[me] = x_ref[pl.ds(me * M_PER, M_PER), :]

        sends = []
        for off in range(1, N_DEV):
            dst = (me + off) % N_DEV
            rdma = pltpu.make_async_remote_copy(
                src_ref=x_ref.at[pl.ds(dst * M_PER, M_PER), :],
                dst_ref=gathered_ref.at[me],
                send_sem=send_sems.at[off],
                recv_sem=recv_sems.at[me],
                device_id=(dst,),
                device_id_type=pl.DeviceIdType.MESH,
            )
            rdma.start()
            sends.append(rdma)

        out_ref[...] = jnp.zeros((M_PER, n), jnp.float32)

        for c in range(N_DEV // CHUNK):
            for j in range(c * CHUNK, (c + 1) * CHUNK):
                recv = pltpu.make_async_remote_copy(
                    src_ref=x_ref.at[pl.ds(0, M_PER), :],
                    dst_ref=gathered_ref.at[j],
                    send_sem=send_sems.at[0],
                    recv_sem=recv_sems.at[j],
                    device_id=(me,),
                    device_id_type=pl.DeviceIdType.MESH,
                )

                @pl.when(j != me)
                def _():
                    recv.wait_recv()

            for j in range(c * CHUNK, (c + 1) * CHUNK):
                out_ref[...] += jnp.dot(
                    gathered_ref[j],
                    w_ref[pl.ds(j * M_PER, M_PER), :],
                    preferred_element_type=jnp.float32,
                )

        for rdma in sends:
            rdma.wait_send()

    return pl.pallas_call(
        body,
        out_shape=jax.ShapeDtypeStruct((M_PER, n), jnp.float32),
        in_specs=[
            pl.BlockSpec(memory_space=pltpu.VMEM),
            pl.BlockSpec(memory_space=pltpu.VMEM),
        ],
        out_specs=pl.BlockSpec(memory_space=pltpu.VMEM),
        scratch_shapes=[
            pltpu.VMEM((N_DEV, M_PER, M_PER), jnp.float32),
            pltpu.SemaphoreType.DMA((N_DEV,)),
            pltpu.SemaphoreType.DMA((N_DEV,)),
        ],
        compiler_params=pltpu.CompilerParams(
            vmem_limit_bytes=100 * 1024 * 1024,
        ),
    )(x, w_mat)


# device time: 45148 ns/iter; 1.2237x vs baseline; 1.2237x over previous
import jax
import jax.numpy as jnp
from jax import lax
from jax.experimental import pallas as pl
from jax.experimental.pallas import tpu as pltpu

N_DEV = 32
NG = 4
NL = 8
M_PER = 128


def kernel(x, w_mat):
    m_glob, k_per = x.shape
    k_glob, n = w_mat.shape

    def body(x_ref, w_ref, out_ref, xb_ref, wb_ref, s_ref, g_ref,
             send_sems1, recv_sems1, send_sems2, recv_sems2):
        me = lax.axis_index("i")
        gg = me // NL
        ll = me % NL

        bar = pltpu.get_barrier_semaphore()
        for off in range(1, NL):
            pl.semaphore_signal(
                bar, inc=1,
                device_id=(gg * NL + (ll + off) % NL,),
                device_id_type=pl.DeviceIdType.MESH,
            )
        for off in range(1, NG):
            pl.semaphore_signal(
                bar, inc=1,
                device_id=(((gg + off) % NG) * NL + ll,),
                device_id_type=pl.DeviceIdType.MESH,
            )
        pl.semaphore_wait(bar, (NL - 1) + (NG - 1))

        xb_ref[...] = x_ref[...].astype(jnp.bfloat16).reshape(NG, NL, M_PER, M_PER)

        s_ref[:, ll] = xb_ref[:, ll]
        sends1 = []
        for off in range(1, NL):
            lp = (ll + off) % NL
            rdma = pltpu.make_async_remote_copy(
                src_ref=xb_ref.at[:, lp],
                dst_ref=s_ref.at[:, ll],
                send_sem=send_sems1.at[off],
                recv_sem=recv_sems1.at[ll],
                device_id=(gg * NL + lp,),
                device_id_type=pl.DeviceIdType.MESH,
            )
            rdma.start()
            sends1.append(rdma)

        wb_ref[...] = w_ref[...].astype(jnp.bfloat16)
        out_ref[...] = jnp.zeros((M_PER, n), jnp.float32)

        for off in range(1, NL):
            ls = (ll + off) % NL
            recv = pltpu.make_async_remote_copy(
                src_ref=xb_ref.at[:, ls],
                dst_ref=s_ref.at[:, ls],
                send_sem=send_sems1.at[off],
                recv_sem=recv_sems1.at[ls],
                device_id=(me,),
                device_id_type=pl.DeviceIdType.MESH,
            )
            recv.wait_recv()

        sends2 = []
        for off in range(1, NG):
            gp = (gg + off) % NG
            rdma = pltpu.make_async_remote_copy(
                src_ref=s_ref.at[gp],
                dst_ref=g_ref.at[gg],
                send_sem=send_sems2.at[off],
                recv_sem=recv_sems2.at[gg],
                device_id=(gp * NL + ll,),
                device_id_type=pl.DeviceIdType.MESH,
            )
            rdma.start()
            sends2.append(rdma)

        g_ref[gg] = s_ref[gg]

        def gemm_chunk(g_src):
            for ls in range(NL):
                j = g_src * NL + ls
                out_ref[...] += jnp.dot(
                    g_ref[g_src, ls],
                    wb_ref[pl.ds(j * M_PER, M_PER), :],
                    preferred_element_type=jnp.float32,
                )

        gemm_chunk(gg)
        for off in range(1, NG):
            gs = (gg + off) % NG
            recv = pltpu.make_async_remote_copy(
                src_ref=s_ref.at[gs],
                dst_ref=g_ref.at[gs],
                send_sem=send_sems2.at[off],
                recv_sem=recv_sems2.at[gs],
                device_id=(me,),
                device_id_type=pl.DeviceIdType.MESH,
            )
            recv.wait_recv()
            gemm_chunk(gs)

        for rdma in sends1:
            rdma.wait_send()
        for rdma in sends2:
            rdma.wait_send()

    return pl.pallas_call(
        body,
        out_shape=jax.ShapeDtypeStruct((M_PER, n), jnp.float32),
        in_specs=[
            pl.BlockSpec(memory_space=pltpu.VMEM),
            pl.BlockSpec(memory_space=pltpu.VMEM),
        ],
        out_specs=pl.BlockSpec(memory_space=pltpu.VMEM),
        scratch_shapes=[
            pltpu.VMEM((NG, NL, M_PER, M_PER), jnp.bfloat16),
            pltpu.VMEM((k_glob, n), jnp.bfloat16),
            pltpu.VMEM((NG, NL, M_PER, M_PER), jnp.bfloat16),
            pltpu.VMEM((NG, NL, M_PER, M_PER), jnp.bfloat16),
            pltpu.SemaphoreType.DMA((NL,)),
            pltpu.SemaphoreType.DMA((NL,)),
            pltpu.SemaphoreType.DMA((NG,)),
            pltpu.SemaphoreType.DMA((NG,)),
        ],
        compiler_params=pltpu.CompilerParams(
            collective_id=0,
            vmem_limit_bytes=100 * 1024 * 1024,
        ),
    )(x, w_mat)


# device time: 39873 ns/iter; 1.3856x vs baseline; 1.1323x over previous
import jax
import jax.numpy as jnp
from jax import lax
from jax.experimental import pallas as pl
from jax.experimental.pallas import tpu as pltpu

N_DEV = 32
M_PER = 128
CHUNK = 4


def kernel(x, w_mat):
    m_glob, k_per = x.shape
    k_glob, n = w_mat.shape

    def body(x_ref, w_ref, out_ref, xb_ref, wb_ref, gathered_ref,
             send_sems, recv_sems):
        me = lax.axis_index("i")

        bar = pltpu.get_barrier_semaphore()
        for off in range(1, N_DEV):
            pl.semaphore_signal(
                bar, inc=1,
                device_id=((me + off) % N_DEV,),
                device_id_type=pl.DeviceIdType.MESH,
            )

        xb_ref[...] = x_ref[...].astype(jnp.bfloat16).reshape(N_DEV, M_PER, M_PER)
        gathered_ref[me] = xb_ref[me]

        pl.semaphore_wait(bar, N_DEV - 1)

        sends = []
        for off in range(1, N_DEV):
            dst = (me + off) % N_DEV
            rdma = pltpu.make_async_remote_copy(
                src_ref=xb_ref.at[dst],
                dst_ref=gathered_ref.at[me],
                send_sem=send_sems.at[off],
                recv_sem=recv_sems.at[me],
                device_id=(dst,),
                device_id_type=pl.DeviceIdType.MESH,
            )
            rdma.start()
            sends.append(rdma)

        wb_ref[...] = w_ref[...].astype(jnp.bfloat16)
        out_ref[...] = jnp.zeros((M_PER, n), jnp.float32)

        for c in range(N_DEV // CHUNK):
            for j in range(c * CHUNK, (c + 1) * CHUNK):
                recv = pltpu.make_async_remote_copy(
                    src_ref=xb_ref.at[j],
                    dst_ref=gathered_ref.at[j],
                    send_sem=send_sems.at[0],
                    recv_sem=recv_sems.at[j],
                    device_id=(me,),
                    device_id_type=pl.DeviceIdType.MESH,
                )

                @pl.when(j != me)
                def _():
                    recv.wait_recv()

            acc = None
            for j in range(c * CHUNK, (c + 1) * CHUNK):
                d = jnp.dot(
                    gathered_ref[j],
                    wb_ref[pl.ds(j * M_PER, M_PER), :],
                    preferred_element_type=jnp.float32,
                )
                acc = d if acc is None else acc + d
            out_ref[...] += acc

        for rdma in sends:
            rdma.wait_send()

    return pl.pallas_call(
        body,
        out_shape=jax.ShapeDtypeStruct((M_PER, n), jnp.float32),
        in_specs=[
            pl.BlockSpec(memory_space=pltpu.VMEM),
            pl.BlockSpec(memory_space=pltpu.VMEM),
        ],
        out_specs=pl.BlockSpec(memory_space=pltpu.VMEM),
        scratch_shapes=[
            pltpu.VMEM((N_DEV, M_PER, M_PER), jnp.bfloat16),
            pltpu.VMEM((k_glob, n), jnp.bfloat16),
            pltpu.VMEM((N_DEV, M_PER, M_PER), jnp.bfloat16),
            pltpu.SemaphoreType.DMA((N_DEV,)),
            pltpu.SemaphoreType.DMA((N_DEV,)),
        ],
        compiler_params=pltpu.CompilerParams(
            collective_id=0,
            vmem_limit_bytes=100 * 1024 * 1024,
        ),
    )(x, w_mat)


# device time: 38250 ns/iter; 1.4444x vs baseline; 1.0424x over previous
import jax
import jax.numpy as jnp
from jax import lax
from jax.experimental import pallas as pl
from jax.experimental.pallas import tpu as pltpu

N_DEV = 32
M_PER = 128
CHUNK = 4


def kernel(x, w_mat):
    m_glob, k_per = x.shape
    k_glob, n = w_mat.shape

    def body(x_ref, w_ref, out_ref, xb_ref, wb_ref, gathered_ref,
             send_sems, recv_sems, ready_sems):
        me = lax.axis_index("i")

        for off in range(1, N_DEV):
            pl.semaphore_signal(
                ready_sems.at[me], inc=1,
                device_id=((me + off) % N_DEV,),
                device_id_type=pl.DeviceIdType.MESH,
            )
        bar = pltpu.get_barrier_semaphore()
        for off in (1, N_DEV - 1):
            pl.semaphore_signal(
                bar, inc=1,
                device_id=((me + off) % N_DEV,),
                device_id_type=pl.DeviceIdType.MESH,
            )

        xb_ref[...] = x_ref[...].astype(jnp.bfloat16).reshape(N_DEV, M_PER, M_PER)
        gathered_ref[me] = xb_ref[me]

        pl.semaphore_wait(bar, 2)

        sends = []
        for off in range(1, N_DEV):
            dst = (me + off) % N_DEV
            pl.semaphore_wait(ready_sems.at[dst], 1)
            rdma = pltpu.make_async_remote_copy(
                src_ref=xb_ref.at[dst],
                dst_ref=gathered_ref.at[me],
                send_sem=send_sems.at[off],
                recv_sem=recv_sems.at[me],
                device_id=(dst,),
                device_id_type=pl.DeviceIdType.MESH,
            )
            rdma.start()
            sends.append(rdma)

        wb_ref[...] = w_ref[...].astype(jnp.bfloat16)
        out_ref[...] = jnp.zeros((M_PER, n), jnp.float32)

        for c in range(N_DEV // CHUNK):
            for j in range(c * CHUNK, (c + 1) * CHUNK):
                recv = pltpu.make_async_remote_copy(
                    src_ref=xb_ref.at[j],
                    dst_ref=gathered_ref.at[j],
                    send_sem=send_sems.at[0],
                    recv_sem=recv_sems.at[j],
                    device_id=(me,),
                    device_id_type=pl.DeviceIdType.MESH,
                )

                @pl.when(j != me)
                def _():
                    recv.wait_recv()

            acc = None
            for j in range(c * CHUNK, (c + 1) * CHUNK):
                d = jnp.dot(
                    gathered_ref[j],
                    wb_ref[pl.ds(j * M_PER, M_PER), :],
                    preferred_element_type=jnp.float32,
                )
                acc = d if acc is None else acc + d
            out_ref[...] += acc

        for rdma in sends:
            rdma.wait_send()

    return pl.pallas_call(
        body,
        out_shape=jax.ShapeDtypeStruct((M_PER, n), jnp.float32),
        in_specs=[
            pl.BlockSpec(memory_space=pltpu.VMEM),
            pl.BlockSpec(memory_space=pltpu.VMEM),
        ],
        out_specs=pl.BlockSpec(memory_space=pltpu.VMEM),
        scratch_shapes=[
            pltpu.VMEM((N_DEV, M_PER, M_PER), jnp.bfloat16),
            pltpu.VMEM((k_glob, n), jnp.bfloat16),
            pltpu.VMEM((N_DEV, M_PER, M_PER), jnp.bfloat16),
            pltpu.SemaphoreType.DMA((N_DEV,)),
            pltpu.SemaphoreType.DMA((N_DEV,)),
            pltpu.SemaphoreType.REGULAR((N_DEV,)),
        ],
        compiler_params=pltpu.CompilerParams(
            collective_id=0,
            vmem_limit_bytes=100 * 1024 * 1024,
        ),
    )(x, w_mat)
